# Initial kernel scaffold; baseline (speedup 1.0000x reference)
import functools

import jax
import jax.numpy as jnp
from jax import lax
from jax.experimental import pallas as pl
from jax.experimental.pallas import tpu as pltpu
from jax.experimental.pallas import tpu_sc as plsc

N = 10000
E = 320000
NR = 4
CH = 128
NPASS = 4
PW = 32
CHUNK = 128
NCHUNKS = E // CHUNK
ROWS_PER_TILE = (NR * N) // 16
ZROWS = 250

_f32 = jnp.float32
_i32 = jnp.int32


def _sc_body(with_counts, *refs):
    if with_counts:
        (yflat, srcA, dstA, etA, s_out, cnt_out,
         acc, srcb, dstb, etb, gidxb, sidxb, rows, ones, zbuf, sem) = refs
    else:
        (yflat, srcA, dstA, etA, s_out,
         acc, srcb, dstb, etb, gidxb, sidxb, rows, ones, zbuf, sem) = refs

    c = lax.axis_index("c")
    s = lax.axis_index("s")
    wid = c * 16 + s
    row0 = s * ROWS_PER_TILE

    def _fill(i, _):
        zbuf[i, pl.ds(0, 16)] = jnp.zeros((16,), _f32)
        zbuf[i, pl.ds(16, 16)] = jnp.zeros((16,), _f32)
        return 0
    lax.fori_loop(0, ZROWS, _fill, 0)
    if with_counts:
        def _fill1(i, _):
            ones[i, pl.ds(0, 16)] = jnp.ones((16,), _f32)
            ones[i, pl.ds(16, 16)] = jnp.ones((16,), _f32)
            return 0
        lax.fori_loop(0, CHUNK, _fill1, 0)

    def _zero_acc():
        for k in range(ROWS_PER_TILE // ZROWS):
            pltpu.sync_copy(zbuf, acc.at[pl.ds(row0 + k * ZROWS, ZROWS)])

    _zero_acc()
    plsc.subcore_barrier()

    if with_counts:
        cbase = wid * (NCHUNKS // 32) + jnp.minimum(wid, NCHUNKS % 32)
        cn = (NCHUNKS // 32) + (wid < NCHUNKS % 32).astype(_i32)

        def _cbody(g, _):
            e0 = (cbase + g) * CHUNK
            pltpu.sync_copy(dstA.at[pl.ds(e0, CHUNK)], dstb)
            pltpu.sync_copy(etA.at[pl.ds(e0, CHUNK)], etb)
            for jj in range(CHUNK // 16):
                sl = pl.ds(jj * 16, 16)
                sidxb[sl] = etb[sl] * N + dstb[sl]
            pltpu.sync_copy(ones, acc.at[sidxb], add=True)
            return 0
        lax.fori_loop(0, cn, _cbody, 0)
        plsc.subcore_barrier()
        pltpu.sync_copy(acc.at[pl.ds(row0, ROWS_PER_TILE)],
                        cnt_out.at[pl.ds(c * (NR * N) + row0, ROWS_PER_TILE)])
        _zero_acc()
        plsc.subcore_barrier()

    mbase = s * (NCHUNKS // 16) + jnp.minimum(s, NCHUNKS % 16)
    mn = (NCHUNKS // 16) + (s < NCHUNKS % 16).astype(_i32)

    for j in range(2):
        p = 2 * c + j
        pbase = p * (NR * N)

        def _mbody(g, _):
            e0 = (mbase + g) * CHUNK
            pltpu.sync_copy(srcA.at[pl.ds(e0, CHUNK)], srcb)
            pltpu.sync_copy(dstA.at[pl.ds(e0, CHUNK)], dstb)
            pltpu.sync_copy(etA.at[pl.ds(e0, CHUNK)], etb)
            for jj in range(CHUNK // 16):
                sl = pl.ds(jj * 16, 16)
                t = etb[sl] * N
                gidxb[sl] = pbase + t + srcb[sl]
                sidxb[sl] = t + dstb[sl]
            pltpu.async_copy(yflat.at[gidxb], rows, sem).wait()
            pltpu.sync_copy(rows, acc.at[sidxb], add=True)
            return 0
        lax.fori_loop(0, mn, _mbody, 0)
        plsc.subcore_barrier()
        pltpu.sync_copy(acc.at[pl.ds(row0, ROWS_PER_TILE)],
                        s_out.at[pl.ds(pbase + row0, ROWS_PER_TILE)])
        if j == 0:
            _zero_acc()
            plsc.subcore_barrier()


def _make_sc(with_counts):
    outs = [jax.ShapeDtypeStruct((NPASS * NR * N, PW), _f32)]
    if with_counts:
        outs.append(jax.ShapeDtypeStruct((2 * NR * N, PW), _f32))
    scratch = [
        pltpu.VMEM_SHARED((NR * N, PW), _f32),
        pltpu.VMEM((CHUNK,), _i32),
        pltpu.VMEM((CHUNK,), _i32),
        pltpu.VMEM((CHUNK,), _i32),
        pltpu.VMEM((CHUNK,), _i32),
        pltpu.VMEM((CHUNK,), _i32),
        pltpu.VMEM((CHUNK, PW), _f32),
        pltpu.VMEM((CHUNK, PW), _f32),
        pltpu.VMEM((ZROWS, PW), _f32),
        pltpu.SemaphoreType.DMA,
    ]
    return pl.kernel(
        functools.partial(_sc_body, with_counts),
        out_type=tuple(outs) if with_counts else outs[0],
        mesh=plsc.VectorSubcoreMesh(core_axis_name="c", subcore_axis_name="s"),
        scratch_types=scratch,
    )


BN = 1000
GRID = N // BN


def _mm_body(h_ref, w_ref, y_ref, r_ref):
    res = jnp.dot(h_ref[...], w_ref[...], preferred_element_type=_f32)
    r_ref[...] = res[:, NR * CH:]
    for p in range(NPASS):
        for r in range(NR):
            y_ref[p, r] = res[:, r * CH + p * PW: r * CH + (p + 1) * PW]


def _comb_cols(s_ref, cp_ref):
    cnt = cp_ref[0] + cp_ref[1]
    inv = 1.0 / jnp.maximum(cnt, 1.0)
    cols = []
    for p in range(NPASS):
        a = inv[0][:, 0:1] * s_ref[p, 0]
        for r in range(1, NR):
            a = a + inv[r][:, 0:1] * s_ref[p, r]
        cols.append(a)
    return jnp.concatenate(cols, axis=1)


def _comb_body(s_ref, cp_ref, r0_ref, b_ref, w_ref, y_ref, r_ref):
    h = r0_ref[...] + b_ref[...] + _comb_cols(s_ref, cp_ref)
    h = jnp.maximum(h, 0.0)
    res = jnp.dot(h, w_ref[...], preferred_element_type=_f32)
    r_ref[...] = res[:, NR * CH:]
    for p in range(NPASS):
        for r in range(NR):
            y_ref[p, r] = res[:, r * CH + p * PW: r * CH + (p + 1) * PW]


def _fin_body(s_ref, cp_ref, r1_ref, b_ref, lw_ref, lb_ref, o_ref):
    h = r1_ref[...] + b_ref[...] + _comb_cols(s_ref, cp_ref)
    g = jnp.sum(h, axis=0, keepdims=True)
    part = jnp.dot(g, lw_ref[...], preferred_element_type=_f32)

    @pl.when(pl.program_id(0) == 0)
    def _():
        o_ref[...] = lb_ref[...]
    o_ref[...] += part


def _mm_call(h, wcat):
    return pl.pallas_call(
        _mm_body,
        grid=(GRID,),
        in_specs=[
            pl.BlockSpec((BN, CH), lambda i: (i, 0)),
            pl.BlockSpec((CH, (NR + 1) * CH), lambda i: (0, 0)),
        ],
        out_specs=[
            pl.BlockSpec((NPASS, NR, BN, PW), lambda i: (0, 0, i, 0)),
            pl.BlockSpec((BN, CH), lambda i: (i, 0)),
        ],
        out_shape=[
            jax.ShapeDtypeStruct((NPASS, NR, N, PW), _f32),
            jax.ShapeDtypeStruct((N, CH), _f32),
        ],
    )(h, wcat)


def _comb_call(s4, cp, r0, b, wcat):
    return pl.pallas_call(
        _comb_body,
        grid=(GRID,),
        in_specs=[
            pl.BlockSpec((NPASS, NR, BN, PW), lambda i: (0, 0, i, 0)),
            pl.BlockSpec((2, NR, BN, PW), lambda i: (0, 0, i, 0)),
            pl.BlockSpec((BN, CH), lambda i: (i, 0)),
            pl.BlockSpec((1, CH), lambda i: (0, 0)),
            pl.BlockSpec((CH, (NR + 1) * CH), lambda i: (0, 0)),
        ],
        out_specs=[
            pl.BlockSpec((NPASS, NR, BN, PW), lambda i: (0, 0, i, 0)),
            pl.BlockSpec((BN, CH), lambda i: (i, 0)),
        ],
        out_shape=[
            jax.ShapeDtypeStruct((NPASS, NR, N, PW), _f32),
            jax.ShapeDtypeStruct((N, CH), _f32),
        ],
    )(s4, cp, r0, b, wcat)


def _fin_call(s4, cp, r1, b, lw, lb):
    return pl.pallas_call(
        _fin_body,
        grid=(GRID,),
        in_specs=[
            pl.BlockSpec((NPASS, NR, BN, PW), lambda i: (0, 0, i, 0)),
            pl.BlockSpec((2, NR, BN, PW), lambda i: (0, 0, i, 0)),
            pl.BlockSpec((BN, CH), lambda i: (i, 0)),
            pl.BlockSpec((1, CH), lambda i: (0, 0)),
            pl.BlockSpec((CH, 2), lambda i: (0, 0)),
            pl.BlockSpec((1, 2), lambda i: (0, 0)),
        ],
        out_specs=pl.BlockSpec((1, 2), lambda i: (0, 0)),
        out_shape=jax.ShapeDtypeStruct((1, 2), _f32),
    )(s4, cp, r1, b, lw, lb)


def kernel(x, edge_index, edge_type, edge_attr, W0, root0, b0, W1, root1, b1,
           lin_W, lin_b):
    src = edge_index[0].astype(_i32)
    dst = edge_index[1].astype(_i32)
    et = edge_type.astype(_i32)

    wcat0 = jnp.concatenate([W0[0], W0[1], W0[2], W0[3], root0], axis=1)
    wcat1 = jnp.concatenate([W1[0], W1[1], W1[2], W1[3], root1], axis=1)

    sc0 = _make_sc(True)
    sc1 = _make_sc(False)

    y0, r0 = _mm_call(x, wcat0)
    s0, cntp = sc0(y0.reshape(NPASS * NR * N, PW), src, dst, et)
    s0 = s0.reshape(NPASS, NR, N, PW)
    cp = cntp.reshape(2, NR, N, PW)
    y1, r1 = _comb_call(s0, cp, r0, b0.reshape(1, CH), wcat1)
    s1 = sc1(y1.reshape(NPASS * NR * N, PW), src, dst, et)
    s1 = s1.reshape(NPASS, NR, N, PW)
    return _fin_call(s1, cp, r1, b1.reshape(1, CH), lin_W, lin_b.reshape(1, 2))


# speedup vs baseline: 4.3620x; 4.3620x over previous
import functools

import jax
import jax.numpy as jnp
from jax import lax
from jax.experimental import pallas as pl
from jax.experimental.pallas import tpu as pltpu
from jax.experimental.pallas import tpu_sc as plsc

N = 10000
NP = 10048
E = 320000
NR = 4
CH = 128
NPASS = 4
PW = 32
CHUNK = 128
NCHUNKS = E // CHUNK
R4 = NR * NP
ROWS_PER_TILE = R4 // 16
ZROWS = 1256

_f32 = jnp.float32
_i32 = jnp.int32


def _sc_body(with_counts, *refs):
    if with_counts:
        (yflat, srcA, dstA, etA, s_out, cnt_out,
         acc, srcb, dstb, etb, gidxb, sidxb, rows, ones, zbuf, sem) = refs
    else:
        (yflat, srcA, dstA, etA, s_out,
         acc, srcb, dstb, etb, gidxb, sidxb, rows, ones, zbuf, sem) = refs

    c = lax.axis_index("c")
    s = lax.axis_index("s")
    wid = c * 16 + s
    row0 = s * ROWS_PER_TILE

    def _fill(i, _):
        zbuf[i, pl.ds(0, 16)] = jnp.zeros((16,), _f32)
        zbuf[i, pl.ds(16, 16)] = jnp.zeros((16,), _f32)
        return 0
    lax.fori_loop(0, ZROWS, _fill, 0)
    if with_counts:
        def _fill1(i, _):
            ones[i, pl.ds(0, 16)] = jnp.ones((16,), _f32)
            ones[i, pl.ds(16, 16)] = jnp.ones((16,), _f32)
            return 0
        lax.fori_loop(0, CHUNK, _fill1, 0)

    def _zero_acc():
        for k in range(ROWS_PER_TILE // ZROWS):
            pltpu.sync_copy(zbuf, acc.at[pl.ds(row0 + k * ZROWS, ZROWS)])

    _zero_acc()
    plsc.subcore_barrier()

    if with_counts:
        cbase = wid * (NCHUNKS // 32) + jnp.minimum(wid, NCHUNKS % 32)
        cn = (NCHUNKS // 32) + (wid < NCHUNKS % 32).astype(_i32)

        def _cbody(g, _):
            e0 = (cbase + g) * CHUNK
            pltpu.sync_copy(dstA.at[pl.ds(e0, CHUNK)], dstb)
            pltpu.sync_copy(etA.at[pl.ds(e0, CHUNK)], etb)
            for jj in range(CHUNK // 16):
                sl = pl.ds(jj * 16, 16)
                sidxb[sl] = etb[sl] * NP + dstb[sl]
            pltpu.sync_copy(ones, acc.at[sidxb], add=True)
            return 0
        lax.fori_loop(0, cn, _cbody, 0)
        plsc.subcore_barrier()
        pltpu.sync_copy(acc.at[pl.ds(row0, ROWS_PER_TILE)],
                        cnt_out.at[pl.ds(c * R4 + row0, ROWS_PER_TILE)])
        _zero_acc()
        plsc.subcore_barrier()

    mbase = s * (NCHUNKS // 16) + jnp.minimum(s, NCHUNKS % 16)
    mn = (NCHUNKS // 16) + (s < NCHUNKS % 16).astype(_i32)

    for j in range(2):
        p = 2 * c + j
        pbase = p * R4

        def _mbody(g, _):
            e0 = (mbase + g) * CHUNK
            pltpu.sync_copy(srcA.at[pl.ds(e0, CHUNK)], srcb)
            pltpu.sync_copy(dstA.at[pl.ds(e0, CHUNK)], dstb)
            pltpu.sync_copy(etA.at[pl.ds(e0, CHUNK)], etb)
            for jj in range(CHUNK // 16):
                sl = pl.ds(jj * 16, 16)
                t = etb[sl] * NP
                gidxb[sl] = pbase + t + srcb[sl]
                sidxb[sl] = t + dstb[sl]
            pltpu.async_copy(yflat.at[gidxb], rows, sem).wait()
            pltpu.sync_copy(rows, acc.at[sidxb], add=True)
            return 0
        lax.fori_loop(0, mn, _mbody, 0)
        plsc.subcore_barrier()
        pltpu.sync_copy(acc.at[pl.ds(row0, ROWS_PER_TILE)],
                        s_out.at[pl.ds(pbase + row0, ROWS_PER_TILE)])
        if j == 0:
            _zero_acc()
            plsc.subcore_barrier()


def _make_sc(with_counts):
    outs = [jax.ShapeDtypeStruct((NPASS * R4, PW), _f32)]
    if with_counts:
        outs.append(jax.ShapeDtypeStruct((2 * R4, PW), _f32))
    scratch = [
        pltpu.VMEM_SHARED((R4, PW), _f32),
        pltpu.VMEM((CHUNK,), _i32),
        pltpu.VMEM((CHUNK,), _i32),
        pltpu.VMEM((CHUNK,), _i32),
        pltpu.VMEM((CHUNK,), _i32),
        pltpu.VMEM((CHUNK,), _i32),
        pltpu.VMEM((CHUNK, PW), _f32),
        pltpu.VMEM((CHUNK, PW), _f32),
        pltpu.VMEM((ZROWS, PW), _f32),
        pltpu.SemaphoreType.DMA,
    ]
    return pl.kernel(
        functools.partial(_sc_body, with_counts),
        out_type=tuple(outs) if with_counts else outs[0],
        mesh=plsc.VectorSubcoreMesh(core_axis_name="c", subcore_axis_name="s"),
        scratch_types=scratch,
        compiler_params=pltpu.CompilerParams(use_tc_tiling_on_sc=False),
    )


BN = 1000
GRID = N // BN


def _mm_body(h_ref, w_ref, y_ref, r_ref):
    res = jnp.dot(h_ref[...], w_ref[...], preferred_element_type=_f32)
    r_ref[...] = res[:, NR * CH:]
    for p in range(NPASS):
        for r in range(NR):
            y_ref[p, r] = res[:, r * CH + p * PW: r * CH + (p + 1) * PW]


def _comb_cols(s_ref, cp_ref):
    cnt = cp_ref[0] + cp_ref[1]
    inv = 1.0 / jnp.maximum(cnt, 1.0)
    cols = []
    for p in range(NPASS):
        a = inv[0][:, 0:1] * s_ref[p, 0]
        for r in range(1, NR):
            a = a + inv[r][:, 0:1] * s_ref[p, r]
        cols.append(a)
    return jnp.concatenate(cols, axis=1)


def _comb_body(s_ref, cp_ref, r0_ref, b_ref, w_ref, y_ref, r_ref):
    h = r0_ref[...] + b_ref[...] + _comb_cols(s_ref, cp_ref)
    h = jnp.maximum(h, 0.0)
    res = jnp.dot(h, w_ref[...], preferred_element_type=_f32)
    r_ref[...] = res[:, NR * CH:]
    for p in range(NPASS):
        for r in range(NR):
            y_ref[p, r] = res[:, r * CH + p * PW: r * CH + (p + 1) * PW]


def _fin_body(s_ref, cp_ref, r1_ref, b_ref, lw_ref, lb_ref, o_ref):
    h = r1_ref[...] + b_ref[...] + _comb_cols(s_ref, cp_ref)
    g = jnp.sum(h, axis=0, keepdims=True)
    part = jnp.dot(g, lw_ref[...], preferred_element_type=_f32)

    @pl.when(pl.program_id(0) == 0)
    def _():
        o_ref[...] = lb_ref[...]
    o_ref[...] += part


def _mm_call(h, wcat):
    return pl.pallas_call(
        _mm_body,
        grid=(GRID,),
        in_specs=[
            pl.BlockSpec((BN, CH), lambda i: (i, 0)),
            pl.BlockSpec((CH, (NR + 1) * CH), lambda i: (0, 0)),
        ],
        out_specs=[
            pl.BlockSpec((NPASS, NR, BN, PW), lambda i: (0, 0, i, 0)),
            pl.BlockSpec((BN, CH), lambda i: (i, 0)),
        ],
        out_shape=[
            jax.ShapeDtypeStruct((NPASS, NR, NP, PW), _f32),
            jax.ShapeDtypeStruct((N, CH), _f32),
        ],
    )(h, wcat)


def _comb_call(s4, cp, r0, b, wcat):
    return pl.pallas_call(
        _comb_body,
        grid=(GRID,),
        in_specs=[
            pl.BlockSpec((NPASS, NR, BN, PW), lambda i: (0, 0, i, 0)),
            pl.BlockSpec((2, NR, BN, PW), lambda i: (0, 0, i, 0)),
            pl.BlockSpec((BN, CH), lambda i: (i, 0)),
            pl.BlockSpec((1, CH), lambda i: (0, 0)),
            pl.BlockSpec((CH, (NR + 1) * CH), lambda i: (0, 0)),
        ],
        out_specs=[
            pl.BlockSpec((NPASS, NR, BN, PW), lambda i: (0, 0, i, 0)),
            pl.BlockSpec((BN, CH), lambda i: (i, 0)),
        ],
        out_shape=[
            jax.ShapeDtypeStruct((NPASS, NR, NP, PW), _f32),
            jax.ShapeDtypeStruct((N, CH), _f32),
        ],
    )(s4, cp, r0, b, wcat)


def _fin_call(s4, cp, r1, b, lw, lb):
    return pl.pallas_call(
        _fin_body,
        grid=(GRID,),
        in_specs=[
            pl.BlockSpec((NPASS, NR, BN, PW), lambda i: (0, 0, i, 0)),
            pl.BlockSpec((2, NR, BN, PW), lambda i: (0, 0, i, 0)),
            pl.BlockSpec((BN, CH), lambda i: (i, 0)),
            pl.BlockSpec((1, CH), lambda i: (0, 0)),
            pl.BlockSpec((CH, 2), lambda i: (0, 0)),
            pl.BlockSpec((1, 2), lambda i: (0, 0)),
        ],
        out_specs=pl.BlockSpec((1, 2), lambda i: (0, 0)),
        out_shape=jax.ShapeDtypeStruct((1, 2), _f32),
    )(s4, cp, r1, b, lw, lb)


def kernel(x, edge_index, edge_type, edge_attr, W0, root0, b0, W1, root1, b1,
           lin_W, lin_b):
    src = edge_index[0].astype(_i32)
    dst = edge_index[1].astype(_i32)
    et = edge_type.astype(_i32)

    wcat0 = jnp.concatenate([W0[0], W0[1], W0[2], W0[3], root0], axis=1)
    wcat1 = jnp.concatenate([W1[0], W1[1], W1[2], W1[3], root1], axis=1)

    sc0 = _make_sc(True)
    sc1 = _make_sc(False)

    y0, r0 = _mm_call(x, wcat0)
    s0, cntp = sc0(y0.reshape(NPASS * R4, PW), src, dst, et)
    s0 = s0.reshape(NPASS, NR, NP, PW)
    cp = cntp.reshape(2, NR, NP, PW)
    y1, r1 = _comb_call(s0, cp, r0, b0.reshape(1, CH), wcat1)
    s1 = sc1(y1.reshape(NPASS * R4, PW), src, dst, et)
    s1 = s1.reshape(NPASS, NR, NP, PW)
    return _fin_call(s1, cp, r1, b1.reshape(1, CH), lin_W, lin_b.reshape(1, 2))


# speedup vs baseline: 9.1551x; 2.0988x over previous
import functools

import jax
import jax.numpy as jnp
from jax import lax
from jax.experimental import pallas as pl
from jax.experimental.pallas import tpu as pltpu
from jax.experimental.pallas import tpu_sc as plsc

N = 10000
NP = 10048
E = 320000
NR = 4
CH = 128
NPASS = 4
PW = 32
CHUNK = 128
EG = CHUNK // 16
NCHUNKS = E // CHUNK
R4 = NR * NP
ROWS_PER_TILE = R4 // 16

_f32 = jnp.float32
_i32 = jnp.int32


def _sc_body(with_counts, *refs):
    if with_counts:
        (yflat, ep, zin, s_out, cnt_out,
         acc, eb0, eb1, gi0, gi1, si0, si1, rows0, rows1, ones,
         es0, es1, gs0, gs1, ss0, ss1) = refs
    else:
        (yflat, ep, zin, s_out,
         acc, eb0, eb1, gi0, gi1, si0, si1, rows0, rows1, ones,
         es0, es1, gs0, gs1, ss0, ss1) = refs

    c = lax.axis_index("c")
    s = lax.axis_index("s")
    wid = c * 16 + s
    row0 = s * ROWS_PER_TILE

    if with_counts:
        def _fill1(i, _):
            ones[i, pl.ds(0, 16)] = jnp.ones((16,), _f32)
            ones[i, pl.ds(16, 16)] = jnp.ones((16,), _f32)
            return 0
        lax.fori_loop(0, CHUNK, _fill1, 0)

    def _zero_acc():
        pltpu.sync_copy(zin, acc.at[pl.ds(row0, ROWS_PER_TILE)])

    def eload(ch, eb, sem):
        pltpu.async_copy(ep.at[pl.ds(ch * EG, EG)], eb, sem)

    def ewait(eb, sem):
        pltpu.make_async_copy(ep.at[pl.ds(0, EG)], eb, sem).wait()

    def cidx(eb, gi, si, pbase):
        for j in range(EG):
            tv = eb[j, 2] * NP
            if gi is not None:
                gi[pl.ds(j * 16, 16)] = pbase + tv + eb[j, 0]
            si[pl.ds(j * 16, 16)] = tv + eb[j, 1]

    def gstart(gi, rows, sem):
        pltpu.async_copy(yflat.at[gi], rows, sem)

    def gwait(gi, rows, sem):
        pltpu.make_async_copy(yflat.at[gi], rows, sem).wait()

    def sstart(rows, si, sem):
        pltpu.async_copy(rows, acc.at[si], sem, add=True)

    def swait(rows, si, sem):
        pltpu.make_async_copy(rows, acc.at[si], sem).wait()

    _zero_acc()
    plsc.subcore_barrier()

    if with_counts:
        cbase = wid * (NCHUNKS // 32) + jnp.minimum(wid, NCHUNKS % 32)
        npair = (NCHUNKS // 32) // 2

        eload(cbase, eb0, es0)
        eload(cbase + 1, eb1, es1)

        def _cbody(i, _):
            c0 = cbase + 2 * i
            ewait(eb0, es0)

            @pl.when(i >= 1)
            def _():
                swait(ones, si0, ss0)
            cidx(eb0, None, si0, 0)
            sstart(ones, si0, ss0)
            ewait(eb1, es1)

            @pl.when(i >= 1)
            def _():
                swait(ones, si1, ss1)
            cidx(eb1, None, si1, 0)
            sstart(ones, si1, ss1)

            @pl.when(i + 1 < npair)
            def _():
                eload(c0 + 2, eb0, es0)
                eload(c0 + 3, eb1, es1)
            return 0
        lax.fori_loop(0, npair, _cbody, 0)
        swait(ones, si0, ss0)
        swait(ones, si1, ss1)

        @pl.when(wid < NCHUNKS % 32)
        def _():
            ch = cbase + 2 * npair
            eload(ch, eb0, es0)
            ewait(eb0, es0)
            cidx(eb0, None, si0, 0)
            sstart(ones, si0, ss0)
            swait(ones, si0, ss0)

        plsc.subcore_barrier()
        pltpu.sync_copy(acc.at[pl.ds(row0, ROWS_PER_TILE)],
                        cnt_out.at[pl.ds(c * R4 + row0, ROWS_PER_TILE)])
        _zero_acc()
        plsc.subcore_barrier()

    mbase = s * (NCHUNKS // 16) + jnp.minimum(s, NCHUNKS % 16)
    mpair = (NCHUNKS // 16) // 2

    for j in range(2):
        p = 2 * c + j
        pbase = p * R4

        eload(mbase, eb0, es0)
        eload(mbase + 1, eb1, es1)

        def _mbody(i, _):
            c0 = mbase + 2 * i
            ewait(eb0, es0)

            @pl.when(i >= 1)
            def _():
                swait(rows0, si0, ss0)
            cidx(eb0, gi0, si0, pbase)
            gstart(gi0, rows0, gs0)
            ewait(eb1, es1)

            @pl.when(i >= 1)
            def _():
                swait(rows1, si1, ss1)
            cidx(eb1, gi1, si1, pbase)
            gstart(gi1, rows1, gs1)

            @pl.when(i + 1 < mpair)
            def _():
                eload(c0 + 2, eb0, es0)
                eload(c0 + 3, eb1, es1)

            gwait(gi0, rows0, gs0)
            sstart(rows0, si0, ss0)
            gwait(gi1, rows1, gs1)
            sstart(rows1, si1, ss1)
            return 0
        lax.fori_loop(0, mpair, _mbody, 0)
        swait(rows0, si0, ss0)
        swait(rows1, si1, ss1)

        @pl.when(s < NCHUNKS % 16)
        def _():
            ch = mbase + 2 * mpair
            eload(ch, eb0, es0)
            ewait(eb0, es0)
            cidx(eb0, gi0, si0, pbase)
            gstart(gi0, rows0, gs0)
            gwait(gi0, rows0, gs0)
            sstart(rows0, si0, ss0)
            swait(rows0, si0, ss0)

        plsc.subcore_barrier()
        pltpu.sync_copy(acc.at[pl.ds(row0, ROWS_PER_TILE)],
                        s_out.at[pl.ds(pbase + row0, ROWS_PER_TILE)])
        if j == 0:
            _zero_acc()
            plsc.subcore_barrier()


def _make_sc(with_counts):
    outs = [jax.ShapeDtypeStruct((NPASS * R4, PW), _f32)]
    if with_counts:
        outs.append(jax.ShapeDtypeStruct((2 * R4, PW), _f32))
    scratch = [
        pltpu.VMEM_SHARED((R4, PW), _f32),
        pltpu.VMEM((EG, 3, 16), _i32),
        pltpu.VMEM((EG, 3, 16), _i32),
        pltpu.VMEM((CHUNK,), _i32),
        pltpu.VMEM((CHUNK,), _i32),
        pltpu.VMEM((CHUNK,), _i32),
        pltpu.VMEM((CHUNK,), _i32),
        pltpu.VMEM((CHUNK, PW), _f32),
        pltpu.VMEM((CHUNK, PW), _f32),
        pltpu.VMEM((CHUNK, PW), _f32),
        pltpu.SemaphoreType.DMA,
        pltpu.SemaphoreType.DMA,
        pltpu.SemaphoreType.DMA,
        pltpu.SemaphoreType.DMA,
        pltpu.SemaphoreType.DMA,
        pltpu.SemaphoreType.DMA,
    ]
    return pl.kernel(
        functools.partial(_sc_body, with_counts),
        out_type=tuple(outs) if with_counts else outs[0],
        mesh=plsc.VectorSubcoreMesh(core_axis_name="c", subcore_axis_name="s"),
        scratch_types=scratch,
        compiler_params=pltpu.CompilerParams(use_tc_tiling_on_sc=False),
    )


BN = 1000
GRID = N // BN


def _mm_body(h_ref, w_ref, y_ref, r_ref):
    res = jnp.dot(h_ref[...], w_ref[...], preferred_element_type=_f32)
    r_ref[...] = res[:, NR * CH:]
    for p in range(NPASS):
        for r in range(NR):
            y_ref[p, r] = res[:, r * CH + p * PW: r * CH + (p + 1) * PW]


def _comb_cols(s_ref, cp_ref):
    cnt = cp_ref[0] + cp_ref[1]
    inv = 1.0 / jnp.maximum(cnt, 1.0)
    cols = []
    for p in range(NPASS):
        a = inv[0][:, 0:1] * s_ref[p, 0]
        for r in range(1, NR):
            a = a + inv[r][:, 0:1] * s_ref[p, r]
        cols.append(a)
    return jnp.concatenate(cols, axis=1)


def _comb_body(s_ref, cp_ref, r0_ref, b_ref, w_ref, y_ref, r_ref):
    h = r0_ref[...] + b_ref[...] + _comb_cols(s_ref, cp_ref)
    h = jnp.maximum(h, 0.0)
    res = jnp.dot(h, w_ref[...], preferred_element_type=_f32)
    r_ref[...] = res[:, NR * CH:]
    for p in range(NPASS):
        for r in range(NR):
            y_ref[p, r] = res[:, r * CH + p * PW: r * CH + (p + 1) * PW]


def _fin_body(s_ref, cp_ref, r1_ref, b_ref, lw_ref, lb_ref, o_ref):
    h = r1_ref[...] + b_ref[...] + _comb_cols(s_ref, cp_ref)
    g = jnp.sum(h, axis=0, keepdims=True)
    part = jnp.dot(g, lw_ref[...], preferred_element_type=_f32)

    @pl.when(pl.program_id(0) == 0)
    def _():
        o_ref[...] = lb_ref[...]
    o_ref[...] += part


def _mm_call(h, wcat):
    return pl.pallas_call(
        _mm_body,
        grid=(GRID,),
        in_specs=[
            pl.BlockSpec((BN, CH), lambda i: (i, 0)),
            pl.BlockSpec((CH, (NR + 1) * CH), lambda i: (0, 0)),
        ],
        out_specs=[
            pl.BlockSpec((NPASS, NR, BN, PW), lambda i: (0, 0, i, 0)),
            pl.BlockSpec((BN, CH), lambda i: (i, 0)),
        ],
        out_shape=[
            jax.ShapeDtypeStruct((NPASS, NR, NP, PW), _f32),
            jax.ShapeDtypeStruct((N, CH), _f32),
        ],
    )(h, wcat)


def _comb_call(s4, cp, r0, b, wcat):
    return pl.pallas_call(
        _comb_body,
        grid=(GRID,),
        in_specs=[
            pl.BlockSpec((NPASS, NR, BN, PW), lambda i: (0, 0, i, 0)),
            pl.BlockSpec((2, NR, BN, PW), lambda i: (0, 0, i, 0)),
            pl.BlockSpec((BN, CH), lambda i: (i, 0)),
            pl.BlockSpec((1, CH), lambda i: (0, 0)),
            pl.BlockSpec((CH, (NR + 1) * CH), lambda i: (0, 0)),
        ],
        out_specs=[
            pl.BlockSpec((NPASS, NR, BN, PW), lambda i: (0, 0, i, 0)),
            pl.BlockSpec((BN, CH), lambda i: (i, 0)),
        ],
        out_shape=[
            jax.ShapeDtypeStruct((NPASS, NR, NP, PW), _f32),
            jax.ShapeDtypeStruct((N, CH), _f32),
        ],
    )(s4, cp, r0, b, wcat)


def _fin_call(s4, cp, r1, b, lw, lb):
    return pl.pallas_call(
        _fin_body,
        grid=(GRID,),
        in_specs=[
            pl.BlockSpec((NPASS, NR, BN, PW), lambda i: (0, 0, i, 0)),
            pl.BlockSpec((2, NR, BN, PW), lambda i: (0, 0, i, 0)),
            pl.BlockSpec((BN, CH), lambda i: (i, 0)),
            pl.BlockSpec((1, CH), lambda i: (0, 0)),
            pl.BlockSpec((CH, 2), lambda i: (0, 0)),
            pl.BlockSpec((1, 2), lambda i: (0, 0)),
        ],
        out_specs=pl.BlockSpec((1, 2), lambda i: (0, 0)),
        out_shape=jax.ShapeDtypeStruct((1, 2), _f32),
    )(s4, cp, r1, b, lw, lb)


def kernel(x, edge_index, edge_type, edge_attr, W0, root0, b0, W1, root1, b1,
           lin_W, lin_b):
    src = edge_index[0].astype(_i32)
    dst = edge_index[1].astype(_i32)
    et = edge_type.astype(_i32)
    ep = jnp.stack([src.reshape(-1, 16), dst.reshape(-1, 16),
                    et.reshape(-1, 16)], axis=1)

    wcat0 = jnp.concatenate([W0[0], W0[1], W0[2], W0[3], root0], axis=1)
    wcat1 = jnp.concatenate([W1[0], W1[1], W1[2], W1[3], root1], axis=1)

    sc0 = _make_sc(True)
    sc1 = _make_sc(False)

    y0, r0 = _mm_call(x, wcat0)
    zin = jnp.zeros((ROWS_PER_TILE, PW), _f32)
    s0, cntp = sc0(y0.reshape(NPASS * R4, PW), ep, zin)
    s0 = s0.reshape(NPASS, NR, NP, PW)
    cp = cntp.reshape(2, NR, NP, PW)
    y1, r1 = _comb_call(s0, cp, r0, b0.reshape(1, CH), wcat1)
    s1 = sc1(y1.reshape(NPASS * R4, PW), ep, zin)
    s1 = s1.reshape(NPASS, NR, NP, PW)
    return _fin_call(s1, cp, r1, b1.reshape(1, CH), lin_W, lin_b.reshape(1, 2))


# speedup vs baseline: 13.8706x; 1.5151x over previous
import functools

import jax
import jax.numpy as jnp
from jax import lax
from jax.experimental import pallas as pl
from jax.experimental.pallas import tpu as pltpu
from jax.experimental.pallas import tpu_sc as plsc

N = 10000
NP = 10048
E = 320000
NR = 4
CH = 128
NPASS = 4
PW = 32
CHUNK = 128
EG = CHUNK // 16
NCHUNKS = E // CHUNK
R4 = NR * NP
ROWS_PER_TILE = R4 // 16

_f32 = jnp.float32
_i32 = jnp.int32


def _sc_body(with_counts, *refs):
    if with_counts:
        (yflat, ep, zin, s_out, cnt_out,
         acc, eb0, eb1, gi0, gi1, si0, si1, rows0, rows1, ones,
         es0, es1, gs0, gs1, ss0, ss1) = refs
    else:
        (yflat, ep, zin, s_out,
         acc, eb0, eb1, gi0, gi1, si0, si1, rows0, rows1, ones,
         es0, es1, gs0, gs1, ss0, ss1) = refs

    c = lax.axis_index("c")
    s = lax.axis_index("s")
    wid = c * 16 + s
    row0 = s * ROWS_PER_TILE

    if with_counts:
        def _fill1(i, _):
            ones[i, pl.ds(0, 16)] = jnp.ones((16,), _f32)
            ones[i, pl.ds(16, 16)] = jnp.ones((16,), _f32)
            return 0
        lax.fori_loop(0, CHUNK, _fill1, 0)

    def _zero_acc():
        pltpu.sync_copy(zin, acc.at[pl.ds(row0, ROWS_PER_TILE)])

    def eload(ch, eb, sem):
        pltpu.async_copy(ep.at[ch], eb, sem)

    def ewait(eb, sem):
        pltpu.make_async_copy(ep.at[0], eb, sem).wait()

    def cidx(eb, gi, si, p):
        for j in range(EG):
            o = j * 48
            sv = eb[o // 128, pl.ds(o % 128, 16)]
            dv = eb[(o + 16) // 128, pl.ds((o + 16) % 128, 16)]
            tv = eb[(o + 32) // 128, pl.ds((o + 32) % 128, 16)]
            tvn = tv * NP
            if gi is not None:
                gi[pl.ds(j * 16, 16)] = (tvn + sv) * NPASS + p
            si[pl.ds(j * 16, 16)] = tvn + dv

    def gstart(gi, rows, sem):
        pltpu.async_copy(yflat.at[gi], rows, sem)

    def gwait(gi, rows, sem):
        pltpu.make_async_copy(yflat.at[gi], rows, sem).wait()

    def sstart(rows, si, sem):
        pltpu.async_copy(rows, acc.at[si], sem, add=True)

    def swait(rows, si, sem):
        pltpu.make_async_copy(rows, acc.at[si], sem).wait()

    _zero_acc()
    plsc.subcore_barrier()

    if with_counts:
        cbase = wid * (NCHUNKS // 32) + jnp.minimum(wid, NCHUNKS % 32)
        npair = (NCHUNKS // 32) // 2

        eload(cbase, eb0, es0)
        eload(cbase + 1, eb1, es1)

        def _cbody(i, _):
            c0 = cbase + 2 * i
            ewait(eb0, es0)

            @pl.when(i >= 1)
            def _():
                swait(ones, si0, ss0)
            cidx(eb0, None, si0, 0)
            sstart(ones, si0, ss0)
            ewait(eb1, es1)

            @pl.when(i >= 1)
            def _():
                swait(ones, si1, ss1)
            cidx(eb1, None, si1, 0)
            sstart(ones, si1, ss1)

            @pl.when(i + 1 < npair)
            def _():
                eload(c0 + 2, eb0, es0)
                eload(c0 + 3, eb1, es1)
            return 0
        lax.fori_loop(0, npair, _cbody, 0)
        swait(ones, si0, ss0)
        swait(ones, si1, ss1)

        @pl.when(wid < NCHUNKS % 32)
        def _():
            ch = cbase + 2 * npair
            eload(ch, eb0, es0)
            ewait(eb0, es0)
            cidx(eb0, None, si0, 0)
            sstart(ones, si0, ss0)
            swait(ones, si0, ss0)

        plsc.subcore_barrier()
        pltpu.sync_copy(acc.at[pl.ds(row0, ROWS_PER_TILE)],
                        cnt_out.at[pl.ds(row0, ROWS_PER_TILE), c])
        _zero_acc()
        plsc.subcore_barrier()

    mbase = s * (NCHUNKS // 16) + jnp.minimum(s, NCHUNKS % 16)
    mpair = (NCHUNKS // 16) // 2

    for j in range(2):
        p = 2 * c + j

        eload(mbase, eb0, es0)
        eload(mbase + 1, eb1, es1)

        def _mbody(i, _):
            c0 = mbase + 2 * i
            ewait(eb0, es0)

            @pl.when(i >= 1)
            def _():
                swait(rows0, si0, ss0)
            cidx(eb0, gi0, si0, p)
            gstart(gi0, rows0, gs0)
            ewait(eb1, es1)

            @pl.when(i >= 1)
            def _():
                swait(rows1, si1, ss1)
            cidx(eb1, gi1, si1, p)
            gstart(gi1, rows1, gs1)

            @pl.when(i + 1 < mpair)
            def _():
                eload(c0 + 2, eb0, es0)
                eload(c0 + 3, eb1, es1)

            gwait(gi0, rows0, gs0)
            sstart(rows0, si0, ss0)
            gwait(gi1, rows1, gs1)
            sstart(rows1, si1, ss1)
            return 0
        lax.fori_loop(0, mpair, _mbody, 0)
        swait(rows0, si0, ss0)
        swait(rows1, si1, ss1)

        @pl.when(s < NCHUNKS % 16)
        def _():
            ch = mbase + 2 * mpair
            eload(ch, eb0, es0)
            ewait(eb0, es0)
            cidx(eb0, gi0, si0, p)
            gstart(gi0, rows0, gs0)
            gwait(gi0, rows0, gs0)
            sstart(rows0, si0, ss0)
            swait(rows0, si0, ss0)

        plsc.subcore_barrier()
        pltpu.sync_copy(acc.at[pl.ds(row0, ROWS_PER_TILE)],
                        s_out.at[pl.ds(row0, ROWS_PER_TILE), p])
        if j == 0:
            _zero_acc()
            plsc.subcore_barrier()


def _make_sc(with_counts):
    outs = [jax.ShapeDtypeStruct((R4, NPASS, PW), _f32)]
    if with_counts:
        outs.append(jax.ShapeDtypeStruct((R4, NPASS, PW), _f32))
    scratch = [
        pltpu.VMEM_SHARED((R4, PW), _f32),
        pltpu.VMEM((3, 128), _i32),
        pltpu.VMEM((3, 128), _i32),
        pltpu.VMEM((CHUNK,), _i32),
        pltpu.VMEM((CHUNK,), _i32),
        pltpu.VMEM((CHUNK,), _i32),
        pltpu.VMEM((CHUNK,), _i32),
        pltpu.VMEM((CHUNK, PW), _f32),
        pltpu.VMEM((CHUNK, PW), _f32),
        pltpu.VMEM((CHUNK, PW), _f32),
        pltpu.SemaphoreType.DMA,
        pltpu.SemaphoreType.DMA,
        pltpu.SemaphoreType.DMA,
        pltpu.SemaphoreType.DMA,
        pltpu.SemaphoreType.DMA,
        pltpu.SemaphoreType.DMA,
    ]
    return pl.kernel(
        functools.partial(_sc_body, with_counts),
        out_type=tuple(outs) if with_counts else outs[0],
        mesh=plsc.VectorSubcoreMesh(core_axis_name="c", subcore_axis_name="s"),
        scratch_types=scratch,
        compiler_params=pltpu.CompilerParams(use_tc_tiling_on_sc=False),
    )


BN = 1000
GRID = N // BN


def _mm_body(h_ref, w_ref, y_ref, r_ref):
    res = jnp.dot(h_ref[...], w_ref[...], preferred_element_type=_f32)
    r_ref[...] = res[:, NR * CH:]
    for r in range(NR):
        y_ref[r] = res[:, r * CH:(r + 1) * CH]


def _msg_cols(s_ref, cp_ref):
    cp = cp_ref[...]
    cnt = cp[:, :, 0:PW] + cp[:, :, PW:2 * PW]
    inv = 1.0 / jnp.maximum(cnt, 1.0)
    sv = s_ref[...]
    m = inv[0][:, 0:1] * sv[0]
    for r in range(1, NR):
        m = m + inv[r][:, 0:1] * sv[r]
    return m


def _comb_body(s_ref, cp_ref, r0_ref, b_ref, w_ref, y_ref, r_ref):
    h = r0_ref[...] + b_ref[...] + _msg_cols(s_ref, cp_ref)
    h = jnp.maximum(h, 0.0)
    res = jnp.dot(h, w_ref[...], preferred_element_type=_f32)
    r_ref[...] = res[:, NR * CH:]
    for r in range(NR):
        y_ref[r] = res[:, r * CH:(r + 1) * CH]


def _fin_body(s_ref, cp_ref, r1_ref, b_ref, lw_ref, lb_ref, o_ref):
    h = r1_ref[...] + b_ref[...] + _msg_cols(s_ref, cp_ref)
    g = jnp.sum(h, axis=0, keepdims=True)
    part = jnp.dot(g, lw_ref[...], preferred_element_type=_f32)

    @pl.when(pl.program_id(0) == 0)
    def _():
        o_ref[...] = lb_ref[...]
    o_ref[...] += part


def _mm_call(h, wcat):
    return pl.pallas_call(
        _mm_body,
        grid=(GRID,),
        in_specs=[
            pl.BlockSpec((BN, CH), lambda i: (i, 0)),
            pl.BlockSpec((CH, (NR + 1) * CH), lambda i: (0, 0)),
        ],
        out_specs=[
            pl.BlockSpec((NR, BN, CH), lambda i: (0, i, 0)),
            pl.BlockSpec((BN, CH), lambda i: (i, 0)),
        ],
        out_shape=[
            jax.ShapeDtypeStruct((NR, NP, CH), _f32),
            jax.ShapeDtypeStruct((N, CH), _f32),
        ],
    )(h, wcat)


def _comb_call(s3, cp3, r0, b, wcat):
    return pl.pallas_call(
        _comb_body,
        grid=(GRID,),
        in_specs=[
            pl.BlockSpec((NR, BN, CH), lambda i: (0, i, 0)),
            pl.BlockSpec((NR, BN, CH), lambda i: (0, i, 0)),
            pl.BlockSpec((BN, CH), lambda i: (i, 0)),
            pl.BlockSpec((1, CH), lambda i: (0, 0)),
            pl.BlockSpec((CH, (NR + 1) * CH), lambda i: (0, 0)),
        ],
        out_specs=[
            pl.BlockSpec((NR, BN, CH), lambda i: (0, i, 0)),
            pl.BlockSpec((BN, CH), lambda i: (i, 0)),
        ],
        out_shape=[
            jax.ShapeDtypeStruct((NR, NP, CH), _f32),
            jax.ShapeDtypeStruct((N, CH), _f32),
        ],
    )(s3, cp3, r0, b, wcat)


def _fin_call(s3, cp3, r1, b, lw, lb):
    return pl.pallas_call(
        _fin_body,
        grid=(GRID,),
        in_specs=[
            pl.BlockSpec((NR, BN, CH), lambda i: (0, i, 0)),
            pl.BlockSpec((NR, BN, CH), lambda i: (0, i, 0)),
            pl.BlockSpec((BN, CH), lambda i: (i, 0)),
            pl.BlockSpec((1, CH), lambda i: (0, 0)),
            pl.BlockSpec((CH, 2), lambda i: (0, 0)),
            pl.BlockSpec((1, 2), lambda i: (0, 0)),
        ],
        out_specs=pl.BlockSpec((1, 2), lambda i: (0, 0)),
        out_shape=jax.ShapeDtypeStruct((1, 2), _f32),
    )(s3, cp3, r1, b, lw, lb)


def kernel(x, edge_index, edge_type, edge_attr, W0, root0, b0, W1, root1, b1,
           lin_W, lin_b):
    src = edge_index[0].astype(_i32)
    dst = edge_index[1].astype(_i32)
    et = edge_type.astype(_i32)
    ep = jnp.stack([src.reshape(NCHUNKS, EG, 16), dst.reshape(NCHUNKS, EG, 16),
                    et.reshape(NCHUNKS, EG, 16)], axis=2)
    ep = ep.reshape(NCHUNKS, 3, 128)

    wcat0 = jnp.concatenate([W0[0], W0[1], W0[2], W0[3], root0], axis=1)
    wcat1 = jnp.concatenate([W1[0], W1[1], W1[2], W1[3], root1], axis=1)

    sc0 = _make_sc(True)
    sc1 = _make_sc(False)

    zin = jnp.zeros((ROWS_PER_TILE, PW), _f32)

    y0, r0 = _mm_call(x, wcat0)
    s0, cntp = sc0(y0.reshape(NPASS * R4, PW), ep, zin)
    s3 = s0.reshape(NR, NP, CH)
    cp3 = cntp.reshape(NR, NP, CH)
    y1, r1 = _comb_call(s3, cp3, r0, b0.reshape(1, CH), wcat1)
    s1 = sc1(y1.reshape(NPASS * R4, PW), ep, zin)
    s13 = s1.reshape(NR, NP, CH)
    return _fin_call(s13, cp3, r1, b1.reshape(1, CH), lin_W, lin_b.reshape(1, 2))


# speedup vs baseline: 17.4418x; 1.2575x over previous
import functools

import jax
import jax.numpy as jnp
from jax import lax
from jax.experimental import pallas as pl
from jax.experimental.pallas import tpu as pltpu
from jax.experimental.pallas import tpu_sc as plsc

N = 10000
NP = 10048
E = 320000
NR = 4
CH = 128
NPASS = 4
PW = 32
CHUNK = 128
EG = CHUNK // 16
NCHUNKS = E // CHUNK
R4 = NR * NP
ROWS_PER_TILE = R4 // 16

_f32 = jnp.float32
_i32 = jnp.int32


def _sc_body(with_counts, *refs):
    if with_counts:
        (yflat, ep, zin, s_out, cnt_out, acc,
         eb0, eb1, eb2, eb3, gi0, gi1, gi2, gi3, si0, si1, si2, si3,
         rows0, rows1, rows2, rows3, ones,
         es0, es1, es2, es3, gs0, gs1, gs2, gs3, ss0, ss1, ss2, ss3) = refs
    else:
        (yflat, ep, zin, s_out, acc,
         eb0, eb1, eb2, eb3, gi0, gi1, gi2, gi3, si0, si1, si2, si3,
         rows0, rows1, rows2, rows3, ones,
         es0, es1, es2, es3, gs0, gs1, gs2, gs3, ss0, ss1, ss2, ss3) = refs
    EB = (eb0, eb1, eb2, eb3)
    GI = (gi0, gi1, gi2, gi3)
    SI = (si0, si1, si2, si3)
    RW = (rows0, rows1, rows2, rows3)
    ES = (es0, es1, es2, es3)
    GS = (gs0, gs1, gs2, gs3)
    SS = (ss0, ss1, ss2, ss3)

    c = lax.axis_index("c")
    s = lax.axis_index("s")
    wid = c * 16 + s
    row0 = s * ROWS_PER_TILE

    if with_counts:
        def _fill1(i, _):
            ones[i, pl.ds(0, 16)] = jnp.ones((16,), _f32)
            ones[i, pl.ds(16, 16)] = jnp.ones((16,), _f32)
            return 0
        lax.fori_loop(0, CHUNK, _fill1, 0)

    def _zero_acc():
        pltpu.sync_copy(zin, acc.at[pl.ds(row0, ROWS_PER_TILE)])

    def eload(ch, k):
        pltpu.async_copy(ep.at[pl.ds(ch * 4, 4)], EB[k], ES[k])

    def ewait(k):
        pltpu.make_async_copy(ep.at[pl.ds(0, 4)], EB[k], ES[k]).wait()

    def cidx(k, p, with_g):
        eb, gi, si = EB[k], GI[k], SI[k]
        for j in range(EG):
            sl = pl.ds(j * 16, 16)
            tvn = eb[2, sl] * NP
            if with_g:
                gi[sl] = (tvn + eb[0, sl]) * NPASS + p
            si[sl] = tvn + eb[1, sl]

    def gstart(k):
        pltpu.async_copy(yflat.at[GI[k]], RW[k], GS[k])

    def gwait(k):
        pltpu.make_async_copy(yflat.at[GI[k]], RW[k], GS[k]).wait()

    def sstart(k, rows):
        pltpu.async_copy(rows, acc.at[SI[k]], SS[k], add=True)

    def swait(k, rows):
        pltpu.make_async_copy(rows, acc.at[SI[k]], SS[k]).wait()

    _zero_acc()
    plsc.subcore_barrier()

    if with_counts:
        cbase = wid * (NCHUNKS // 32) + jnp.minimum(wid, NCHUNKS % 32)
        npair = (NCHUNKS // 32) // 2

        eload(cbase, 0)
        eload(cbase + 1, 1)

        def _cbody(i, _):
            c0 = cbase + 2 * i
            for k in range(2):
                ewait(k)

                @pl.when(i >= 1)
                def _():
                    swait(k, ones)
                cidx(k, 0, False)
                sstart(k, ones)

            @pl.when(i + 1 < npair)
            def _():
                eload(c0 + 2, 0)
                eload(c0 + 3, 1)
            return 0
        lax.fori_loop(0, npair, _cbody, 0)
        swait(0, ones)
        swait(1, ones)

        @pl.when(wid < NCHUNKS % 32)
        def _():
            ch = cbase + 2 * npair
            eload(ch, 0)
            ewait(0)
            cidx(0, 0, False)
            sstart(0, ones)
            swait(0, ones)

        plsc.subcore_barrier()
        pltpu.sync_copy(acc.at[pl.ds(row0, ROWS_PER_TILE)],
                        cnt_out.at[pl.ds(row0, ROWS_PER_TILE), c])
        _zero_acc()
        plsc.subcore_barrier()

    mbase = s * (NCHUNKS // 16) + jnp.minimum(s, NCHUNKS % 16)
    nquad = (NCHUNKS // 16) // 4

    for j in range(2):
        p = 2 * c + j

        for k in range(4):
            eload(mbase + k, k)

        def _mbody(i, _):
            q0 = mbase + 4 * i
            for k in range(4):
                @pl.when(i >= 1)
                def _():
                    swait(k, RW[k])
                ewait(k)
                cidx(k, p, True)
                gstart(k)

                @pl.when(i + 1 < nquad)
                def _():
                    eload(q0 + k + 4, k)
                km = (k - 1) % 4
                if k == 0:
                    @pl.when(i >= 1)
                    def _():
                        gwait(km)
                        sstart(km, RW[km])
                else:
                    gwait(km)
                    sstart(km, RW[km])
            return 0
        lax.fori_loop(0, nquad, _mbody, 0)
        gwait(3)
        sstart(3, RW[3])
        for k in range(4):
            swait(k, RW[k])

        @pl.when(s < NCHUNKS % 16)
        def _():
            ch = mbase + 4 * nquad
            eload(ch, 0)
            ewait(0)
            cidx(0, p, True)
            gstart(0)
            gwait(0)
            sstart(0, RW[0])
            swait(0, RW[0])

        plsc.subcore_barrier()
        pltpu.sync_copy(acc.at[pl.ds(row0, ROWS_PER_TILE)],
                        s_out.at[pl.ds(row0, ROWS_PER_TILE), p])
        if j == 0:
            _zero_acc()
            plsc.subcore_barrier()


def _make_sc(with_counts):
    outs = [jax.ShapeDtypeStruct((R4, NPASS, PW), _f32)]
    if with_counts:
        outs.append(jax.ShapeDtypeStruct((R4, NPASS, PW), _f32))
    scratch = (
        [pltpu.VMEM_SHARED((R4, PW), _f32)] +
        [pltpu.VMEM((4, 128), _i32) for _ in range(4)] +
        [pltpu.VMEM((CHUNK,), _i32) for _ in range(8)] +
        [pltpu.VMEM((CHUNK, PW), _f32) for _ in range(4)] +
        [pltpu.VMEM((CHUNK, PW), _f32)] +
        [pltpu.SemaphoreType.DMA for _ in range(12)]
    )
    return pl.kernel(
        functools.partial(_sc_body, with_counts),
        out_type=tuple(outs) if with_counts else outs[0],
        mesh=plsc.VectorSubcoreMesh(core_axis_name="c", subcore_axis_name="s"),
        scratch_types=scratch,
        compiler_params=pltpu.CompilerParams(use_tc_tiling_on_sc=False),
    )


BN = 1000
GRID = N // BN


def _mm_body(h_ref, w_ref, y_ref, r_ref):
    res = jnp.dot(h_ref[...], w_ref[...], preferred_element_type=_f32)
    r_ref[...] = res[:, NR * CH:]
    for r in range(NR):
        y_ref[r] = res[:, r * CH:(r + 1) * CH]


def _msg_cols(s_ref, cp_ref):
    cp = cp_ref[...]
    cnt = cp[:, :, 0:PW] + cp[:, :, PW:2 * PW]
    inv = 1.0 / jnp.maximum(cnt, 1.0)
    sv = s_ref[...]
    m = inv[0][:, 0:1] * sv[0]
    for r in range(1, NR):
        m = m + inv[r][:, 0:1] * sv[r]
    return m


def _comb_body(s_ref, cp_ref, r0_ref, b_ref, w_ref, y_ref, r_ref):
    h = r0_ref[...] + b_ref[...] + _msg_cols(s_ref, cp_ref)
    h = jnp.maximum(h, 0.0)
    res = jnp.dot(h, w_ref[...], preferred_element_type=_f32)
    r_ref[...] = res[:, NR * CH:]
    for r in range(NR):
        y_ref[r] = res[:, r * CH:(r + 1) * CH]


def _fin_body(s_ref, cp_ref, r1_ref, b_ref, lw_ref, lb_ref, o_ref):
    h = r1_ref[...] + b_ref[...] + _msg_cols(s_ref, cp_ref)
    g = jnp.sum(h, axis=0, keepdims=True)
    part = jnp.dot(g, lw_ref[...], preferred_element_type=_f32)

    @pl.when(pl.program_id(0) == 0)
    def _():
        o_ref[...] = lb_ref[...]
    o_ref[...] += part


def _mm_call(h, wcat):
    return pl.pallas_call(
        _mm_body,
        grid=(GRID,),
        in_specs=[
            pl.BlockSpec((BN, CH), lambda i: (i, 0)),
            pl.BlockSpec((CH, (NR + 1) * CH), lambda i: (0, 0)),
        ],
        out_specs=[
            pl.BlockSpec((NR, BN, CH), lambda i: (0, i, 0)),
            pl.BlockSpec((BN, CH), lambda i: (i, 0)),
        ],
        out_shape=[
            jax.ShapeDtypeStruct((NR, NP, CH), _f32),
            jax.ShapeDtypeStruct((N, CH), _f32),
        ],
    )(h, wcat)


def _comb_call(s3, cp3, r0, b, wcat):
    return pl.pallas_call(
        _comb_body,
        grid=(GRID,),
        in_specs=[
            pl.BlockSpec((NR, BN, CH), lambda i: (0, i, 0)),
            pl.BlockSpec((NR, BN, CH), lambda i: (0, i, 0)),
            pl.BlockSpec((BN, CH), lambda i: (i, 0)),
            pl.BlockSpec((1, CH), lambda i: (0, 0)),
            pl.BlockSpec((CH, (NR + 1) * CH), lambda i: (0, 0)),
        ],
        out_specs=[
            pl.BlockSpec((NR, BN, CH), lambda i: (0, i, 0)),
            pl.BlockSpec((BN, CH), lambda i: (i, 0)),
        ],
        out_shape=[
            jax.ShapeDtypeStruct((NR, NP, CH), _f32),
            jax.ShapeDtypeStruct((N, CH), _f32),
        ],
    )(s3, cp3, r0, b, wcat)


def _fin_call(s3, cp3, r1, b, lw, lb):
    return pl.pallas_call(
        _fin_body,
        grid=(GRID,),
        in_specs=[
            pl.BlockSpec((NR, BN, CH), lambda i: (0, i, 0)),
            pl.BlockSpec((NR, BN, CH), lambda i: (0, i, 0)),
            pl.BlockSpec((BN, CH), lambda i: (i, 0)),
            pl.BlockSpec((1, CH), lambda i: (0, 0)),
            pl.BlockSpec((CH, 2), lambda i: (0, 0)),
            pl.BlockSpec((1, 2), lambda i: (0, 0)),
        ],
        out_specs=pl.BlockSpec((1, 2), lambda i: (0, 0)),
        out_shape=jax.ShapeDtypeStruct((1, 2), _f32),
    )(s3, cp3, r1, b, lw, lb)


def kernel(x, edge_index, edge_type, edge_attr, W0, root0, b0, W1, root1, b1,
           lin_W, lin_b):
    src = edge_index[0].astype(_i32)
    dst = edge_index[1].astype(_i32)
    et = edge_type.astype(_i32)
    ep = jnp.stack([src.reshape(NCHUNKS, 128), dst.reshape(NCHUNKS, 128),
                    et.reshape(NCHUNKS, 128), et.reshape(NCHUNKS, 128)],
                   axis=1).reshape(NCHUNKS * 4, 128)

    wcat0 = jnp.concatenate([W0[0], W0[1], W0[2], W0[3], root0], axis=1)
    wcat1 = jnp.concatenate([W1[0], W1[1], W1[2], W1[3], root1], axis=1)

    sc0 = _make_sc(True)
    sc1 = _make_sc(False)

    zin = jnp.zeros((ROWS_PER_TILE, PW), _f32)

    y0, r0 = _mm_call(x, wcat0)
    s0, cntp = sc0(y0.reshape(NPASS * R4, PW), ep, zin)
    s3 = s0.reshape(NR, NP, CH)
    cp3 = cntp.reshape(NR, NP, CH)
    y1, r1 = _comb_call(s3, cp3, r0, b0.reshape(1, CH), wcat1)
    s1 = sc1(y1.reshape(NPASS * R4, PW), ep, zin)
    s13 = s1.reshape(NR, NP, CH)
    return _fin_call(s13, cp3, r1, b1.reshape(1, CH), lin_W, lin_b.reshape(1, 2))


# speedup vs baseline: 17.9810x; 1.0309x over previous
import functools

import jax
import jax.numpy as jnp
from jax import lax
from jax.experimental import pallas as pl
from jax.experimental.pallas import tpu as pltpu
from jax.experimental.pallas import tpu_sc as plsc

N = 10000
NP = 10048
E = 320000
NR = 4
CH = 128
NPASS = 4
PW = 32
CHUNK = 128
EG = CHUNK // 16
NCHUNKS = E // CHUNK
R4 = NR * NP
ROWS_PER_TILE = R4 // 16

_f32 = jnp.float32
_i32 = jnp.int32


def _sc_body(with_counts, *refs):
    if with_counts:
        (yflat, ei, etA, zin, s_out, cnt_out, acc,
         eb0, eb1, eb2, eb3, gi0, gi1, gi2, gi3, si0, si1, si2, si3,
         rows0, rows1, rows2, rows3, ones,
         es0, es1, es2, es3, gs0, gs1, gs2, gs3, ss0, ss1, ss2, ss3) = refs
    else:
        (yflat, ei, etA, zin, s_out, acc,
         eb0, eb1, eb2, eb3, gi0, gi1, gi2, gi3, si0, si1, si2, si3,
         rows0, rows1, rows2, rows3, ones,
         es0, es1, es2, es3, gs0, gs1, gs2, gs3, ss0, ss1, ss2, ss3) = refs
    EB = (eb0, eb1, eb2, eb3)
    GI = (gi0, gi1, gi2, gi3)
    SI = (si0, si1, si2, si3)
    RW = (rows0, rows1, rows2, rows3)
    ES = (es0, es1, es2, es3)
    GS = (gs0, gs1, gs2, gs3)
    SS = (ss0, ss1, ss2, ss3)

    c = lax.axis_index("c")
    s = lax.axis_index("s")
    wid = c * 16 + s
    row0 = s * ROWS_PER_TILE

    if with_counts:
        def _fill1(i, _):
            ones[i, pl.ds(0, 16)] = jnp.ones((16,), _f32)
            ones[i, pl.ds(16, 16)] = jnp.ones((16,), _f32)
            return 0
        lax.fori_loop(0, CHUNK, _fill1, 0)

    def _zero_acc():
        pltpu.sync_copy(zin, acc.at[pl.ds(row0, ROWS_PER_TILE)])

    def eload(ch, k):
        e0 = ch * CHUNK
        pltpu.async_copy(ei.at[0, pl.ds(e0, CHUNK)], EB[k].at[0], ES[k])
        pltpu.async_copy(ei.at[1, pl.ds(e0, CHUNK)], EB[k].at[1], ES[k])
        pltpu.async_copy(etA.at[pl.ds(e0, CHUNK)], EB[k].at[2], ES[k])

    def ewait(k):
        for _ in range(3):
            pltpu.make_async_copy(etA.at[pl.ds(0, CHUNK)], EB[k].at[2],
                                  ES[k]).wait()

    def cidx(k, p, with_g):
        eb, gi, si = EB[k], GI[k], SI[k]
        for j in range(EG):
            sl = pl.ds(j * 16, 16)
            tvn = eb[2, sl] * NP
            if with_g:
                gi[sl] = (tvn + eb[0, sl]) * NPASS + p
            si[sl] = tvn + eb[1, sl]

    def gstart(k):
        pltpu.async_copy(yflat.at[GI[k]], RW[k], GS[k])

    def gwait(k):
        pltpu.make_async_copy(yflat.at[GI[k]], RW[k], GS[k]).wait()

    def sstart(k, rows):
        pltpu.async_copy(rows, acc.at[SI[k]], SS[k], add=True)

    def swait(k, rows):
        pltpu.make_async_copy(rows, acc.at[SI[k]], SS[k]).wait()

    _zero_acc()
    plsc.subcore_barrier()

    if with_counts:
        cbase = wid * (NCHUNKS // 32) + jnp.minimum(wid, NCHUNKS % 32)
        npair = (NCHUNKS // 32) // 2

        eload(cbase, 0)
        eload(cbase + 1, 1)

        def _cbody(i, _):
            c0 = cbase + 2 * i
            for k in range(2):
                ewait(k)

                @pl.when(i >= 1)
                def _():
                    swait(k, ones)
                cidx(k, 0, False)
                sstart(k, ones)

            @pl.when(i + 1 < npair)
            def _():
                eload(c0 + 2, 0)
                eload(c0 + 3, 1)
            return 0
        lax.fori_loop(0, npair, _cbody, 0)
        swait(0, ones)
        swait(1, ones)

        @pl.when(wid < NCHUNKS % 32)
        def _():
            ch = cbase + 2 * npair
            eload(ch, 0)
            ewait(0)
            cidx(0, 0, False)
            sstart(0, ones)
            swait(0, ones)

        plsc.subcore_barrier()
        pltpu.sync_copy(acc.at[pl.ds(row0, ROWS_PER_TILE)],
                        cnt_out.at[pl.ds(row0, ROWS_PER_TILE), c])
        _zero_acc()
        plsc.subcore_barrier()

    mbase = s * (NCHUNKS // 16) + jnp.minimum(s, NCHUNKS % 16)
    nquad = (NCHUNKS // 16) // 4

    for j in range(2):
        p = 2 * c + j

        for k in range(4):
            eload(mbase + k, k)

        def _mbody(i, _):
            q0 = mbase + 4 * i
            for k in range(4):
                @pl.when(i >= 1)
                def _():
                    swait(k, RW[k])
                ewait(k)
                cidx(k, p, True)
                gstart(k)

                @pl.when(i + 1 < nquad)
                def _():
                    eload(q0 + k + 4, k)
                km = (k - 1) % 4
                if k == 0:
                    @pl.when(i >= 1)
                    def _():
                        gwait(km)
                        sstart(km, RW[km])
                else:
                    gwait(km)
                    sstart(km, RW[km])
            return 0
        lax.fori_loop(0, nquad, _mbody, 0)
        gwait(3)
        sstart(3, RW[3])
        for k in range(4):
            swait(k, RW[k])

        @pl.when(s < NCHUNKS % 16)
        def _():
            ch = mbase + 4 * nquad
            eload(ch, 0)
            ewait(0)
            cidx(0, p, True)
            gstart(0)
            gwait(0)
            sstart(0, RW[0])
            swait(0, RW[0])

        plsc.subcore_barrier()
        pltpu.sync_copy(acc.at[pl.ds(row0, ROWS_PER_TILE)],
                        s_out.at[pl.ds(row0, ROWS_PER_TILE), p])
        if j == 0:
            _zero_acc()
            plsc.subcore_barrier()


def _make_sc(with_counts):
    outs = [jax.ShapeDtypeStruct((R4, NPASS, PW), _f32)]
    if with_counts:
        outs.append(jax.ShapeDtypeStruct((R4, NPASS, PW), _f32))
    scratch = (
        [pltpu.VMEM_SHARED((R4, PW), _f32)] +
        [pltpu.VMEM((3, 128), _i32) for _ in range(4)] +
        [pltpu.VMEM((CHUNK,), _i32) for _ in range(8)] +
        [pltpu.VMEM((CHUNK, PW), _f32) for _ in range(4)] +
        [pltpu.VMEM((CHUNK, PW), _f32)] +
        [pltpu.SemaphoreType.DMA for _ in range(12)]
    )
    return pl.kernel(
        functools.partial(_sc_body, with_counts),
        out_type=tuple(outs) if with_counts else outs[0],
        mesh=plsc.VectorSubcoreMesh(core_axis_name="c", subcore_axis_name="s"),
        scratch_types=scratch,
        compiler_params=pltpu.CompilerParams(use_tc_tiling_on_sc=False),
    )


BN = 1000
GRID = N // BN


def _mm_body(h_ref, w_ref, y_ref, r_ref):
    res = jnp.dot(h_ref[...], w_ref[...], preferred_element_type=_f32)
    r_ref[...] = res[:, NR * CH:]
    for r in range(NR):
        y_ref[r] = res[:, r * CH:(r + 1) * CH]


def _msg_cols(s_ref, cp_ref):
    cp = cp_ref[...]
    cnt = cp[:, :, 0:PW] + cp[:, :, PW:2 * PW]
    inv = 1.0 / jnp.maximum(cnt, 1.0)
    sv = s_ref[...]
    m = inv[0][:, 0:1] * sv[0]
    for r in range(1, NR):
        m = m + inv[r][:, 0:1] * sv[r]
    return m


def _comb_body(s_ref, cp_ref, r0_ref, b_ref, w_ref, y_ref, r_ref):
    h = r0_ref[...] + b_ref[...] + _msg_cols(s_ref, cp_ref)
    h = jnp.maximum(h, 0.0)
    res = jnp.dot(h, w_ref[...], preferred_element_type=_f32)
    r_ref[...] = res[:, NR * CH:]
    for r in range(NR):
        y_ref[r] = res[:, r * CH:(r + 1) * CH]


def _fin_body(s_ref, cp_ref, r1_ref, b_ref, lw_ref, lb_ref, o_ref):
    h = r1_ref[...] + b_ref[...] + _msg_cols(s_ref, cp_ref)
    g = jnp.sum(h, axis=0, keepdims=True)
    part = jnp.dot(g, lw_ref[...], preferred_element_type=_f32)

    @pl.when(pl.program_id(0) == 0)
    def _():
        o_ref[...] = lb_ref[...]
    o_ref[...] += part


def _mm_call(h, wcat):
    return pl.pallas_call(
        _mm_body,
        grid=(GRID,),
        in_specs=[
            pl.BlockSpec((BN, CH), lambda i: (i, 0)),
            pl.BlockSpec((CH, (NR + 1) * CH), lambda i: (0, 0)),
        ],
        out_specs=[
            pl.BlockSpec((NR, BN, CH), lambda i: (0, i, 0)),
            pl.BlockSpec((BN, CH), lambda i: (i, 0)),
        ],
        out_shape=[
            jax.ShapeDtypeStruct((NR, NP, CH), _f32),
            jax.ShapeDtypeStruct((N, CH), _f32),
        ],
    )(h, wcat)


def _comb_call(s3, cp3, r0, b, wcat):
    return pl.pallas_call(
        _comb_body,
        grid=(GRID,),
        in_specs=[
            pl.BlockSpec((NR, BN, CH), lambda i: (0, i, 0)),
            pl.BlockSpec((NR, BN, CH), lambda i: (0, i, 0)),
            pl.BlockSpec((BN, CH), lambda i: (i, 0)),
            pl.BlockSpec((1, CH), lambda i: (0, 0)),
            pl.BlockSpec((CH, (NR + 1) * CH), lambda i: (0, 0)),
        ],
        out_specs=[
            pl.BlockSpec((NR, BN, CH), lambda i: (0, i, 0)),
            pl.BlockSpec((BN, CH), lambda i: (i, 0)),
        ],
        out_shape=[
            jax.ShapeDtypeStruct((NR, NP, CH), _f32),
            jax.ShapeDtypeStruct((N, CH), _f32),
        ],
    )(s3, cp3, r0, b, wcat)


def _fin_call(s3, cp3, r1, b, lw, lb):
    return pl.pallas_call(
        _fin_body,
        grid=(GRID,),
        in_specs=[
            pl.BlockSpec((NR, BN, CH), lambda i: (0, i, 0)),
            pl.BlockSpec((NR, BN, CH), lambda i: (0, i, 0)),
            pl.BlockSpec((BN, CH), lambda i: (i, 0)),
            pl.BlockSpec((1, CH), lambda i: (0, 0)),
            pl.BlockSpec((CH, 2), lambda i: (0, 0)),
            pl.BlockSpec((1, 2), lambda i: (0, 0)),
        ],
        out_specs=pl.BlockSpec((1, 2), lambda i: (0, 0)),
        out_shape=jax.ShapeDtypeStruct((1, 2), _f32),
    )(s3, cp3, r1, b, lw, lb)


def kernel(x, edge_index, edge_type, edge_attr, W0, root0, b0, W1, root1, b1,
           lin_W, lin_b):
    et = edge_type.astype(_i32)
    ei = edge_index.astype(_i32)

    wcat0 = jnp.concatenate([W0[0], W0[1], W0[2], W0[3], root0], axis=1)
    wcat1 = jnp.concatenate([W1[0], W1[1], W1[2], W1[3], root1], axis=1)

    sc0 = _make_sc(True)
    sc1 = _make_sc(False)

    zin = jnp.zeros((ROWS_PER_TILE, PW), _f32)

    y0, r0 = _mm_call(x, wcat0)
    s0, cntp = sc0(y0.reshape(NPASS * R4, PW), ei, et, zin)
    s3 = s0.reshape(NR, NP, CH)
    cp3 = cntp.reshape(NR, NP, CH)
    y1, r1 = _comb_call(s3, cp3, r0, b0.reshape(1, CH), wcat1)
    s1 = sc1(y1.reshape(NPASS * R4, PW), ei, et, zin)
    s13 = s1.reshape(NR, NP, CH)
    return _fin_call(s13, cp3, r1, b1.reshape(1, CH), lin_W, lin_b.reshape(1, 2))
